# baseline (speedup 1.0000x reference)
import functools

import jax
import jax.numpy as jnp
from jax import lax
from jax.experimental import pallas as pl
from jax.experimental.pallas import tpu as pltpu
from jax.experimental.pallas import tpu_sc as plsc


def _gather_kernel(B, D, n_workers, num_cores, chunk):
    n_chunks = (B // n_workers) // chunk
    b_per_w = B // n_workers
    mesh = plsc.VectorSubcoreMesh(core_axis_name="c", subcore_axis_name="s")

    @functools.partial(
        pl.kernel,
        mesh=mesh,
        out_type=jax.ShapeDtypeStruct((B, D), jnp.float32),
        scratch_types=[
            pltpu.VMEM((chunk,), jnp.int32),
            pltpu.VMEM((chunk, D), jnp.float32),
            pltpu.SemaphoreType.DMA,
        ],
        compiler_params=pltpu.CompilerParams(use_tc_tiling_on_sc=False),
    )
    def k(table_hbm, idx_hbm, out_hbm, idx_v, rows_v, sem):
        wid = lax.axis_index("s") * num_cores + lax.axis_index("c")
        base = wid * b_per_w

        def body(i, carry):
            off = base + i * chunk
            pltpu.sync_copy(idx_hbm.at[pl.ds(off, chunk)], idx_v)
            pltpu.async_copy(table_hbm.at[idx_v], rows_v, sem).wait()
            pltpu.sync_copy(rows_v, out_hbm.at[pl.ds(off, chunk)])
            return carry

        lax.fori_loop(0, n_chunks, body, 0)

    return k


def kernel(x, mask, table):
    del mask
    batch, seq = x.shape
    _, d_model = table.shape
    idx = x.reshape(-1).astype(jnp.int32)
    B = idx.shape[0]

    info = plsc.get_sparse_core_info()
    n_workers = info.num_cores * info.num_subcores
    chunk = 1024
    assert B % (n_workers * chunk) == 0

    out = _gather_kernel(B, d_model, n_workers, info.num_cores, chunk)(table, idx)
    return out.reshape(batch, seq, d_model)


# speedup vs baseline: 1.0175x; 1.0175x over previous
import functools

import jax
import jax.numpy as jnp
from jax import lax
from jax.experimental import pallas as pl
from jax.experimental.pallas import tpu as pltpu
from jax.experimental.pallas import tpu_sc as plsc


def _gather_kernel(B, D, n_workers, num_cores, chunk, nbuf):
    b_per_w = B // n_workers
    n = b_per_w // chunk
    mesh = plsc.VectorSubcoreMesh(core_axis_name="c", subcore_axis_name="s")

    @functools.partial(
        pl.kernel,
        mesh=mesh,
        out_type=jax.ShapeDtypeStruct((B, D), jnp.float32),
        scratch_types=[
            pltpu.VMEM((b_per_w,), jnp.int32),
            pltpu.VMEM((nbuf, chunk, D), jnp.float32),
            pltpu.SemaphoreType.DMA,
            pltpu.SemaphoreType.DMA,
        ],
        compiler_params=pltpu.CompilerParams(use_tc_tiling_on_sc=False),
    )
    def k(table_hbm, idx_hbm, out_hbm, idx_v, rows_v, gsem, osem):
        wid = lax.axis_index("s") * num_cores + lax.axis_index("c")
        base = wid * b_per_w

        pltpu.sync_copy(idx_hbm.at[pl.ds(base, b_per_w)], idx_v)

        def issue_gather(i, b):
            pltpu.async_copy(
                table_hbm.at[idx_v.at[pl.ds(i * chunk, chunk)]],
                rows_v.at[b],
                gsem,
            )

        for b in range(nbuf - 1):
            issue_gather(b, b)

        def wait_quantum(sem):
            pltpu.make_async_copy(
                out_hbm.at[pl.ds(0, chunk)], rows_v.at[0], sem
            ).wait()

        def step(i, b, issue):
            wait_quantum(gsem)
            pltpu.async_copy(
                rows_v.at[b],
                out_hbm.at[pl.ds(base + i * chunk, chunk)],
                osem,
            )

            @pl.when(i > 0)
            def _():
                wait_quantum(osem)

            if issue:
                issue_gather(i + nbuf - 1, (b + nbuf - 1) % nbuf)

        def outer(t, carry):
            for b in range(nbuf):
                step(t * nbuf + b, b, True)
            return carry

        lax.fori_loop(0, n // nbuf - 1, outer, 0)
        t_last = n // nbuf - 1
        for b in range(nbuf):
            i = t_last * nbuf + b
            step(i, b, issue=(i + nbuf - 1 < n))
        wait_quantum(osem)

    return k


def kernel(x, mask, table):
    del mask
    batch, seq = x.shape
    _, d_model = table.shape
    idx = x.reshape(-1).astype(jnp.int32)
    B = idx.shape[0]

    info = plsc.get_sparse_core_info()
    n_workers = info.num_cores * info.num_subcores
    chunk = 400
    nbuf = 4
    assert B % (n_workers * chunk) == 0
    assert (B // (n_workers * chunk)) % nbuf == 0

    out = _gather_kernel(B, d_model, n_workers, info.num_cores, chunk, nbuf)(
        table, idx
    )
    return out.reshape(batch, seq, d_model)
